# Initial kernel scaffold; baseline (speedup 1.0000x reference)
import jax
import jax.numpy as jnp
from jax.experimental import pallas as pl


def kernel(merged_feats, source_maps):
    raise NotImplementedError("write your pallas kernel here")


# speedup vs baseline: 11.8959x; 11.8959x over previous
import functools

import jax
import jax.numpy as jnp
from jax import lax
from jax.experimental import pallas as pl
from jax.experimental.pallas import tpu as pltpu
from jax.experimental.pallas import tpu_sc as plsc

B, M, L, D = 4, 4096, 8192, 1024
NC, NS, LANES = 2, 16, 16
NW = NC * NS
TPB = NW // B
RPT = L // TPB
CHUNK = 32
NCH = RPT // CHUNK


def _unmerge_body(merged_hbm, idx_hbm, out_hbm,
                  idxb_v, scale_v, gidx_v, scl_v, rows_v,
                  gsem0, gsem1, osem0, osem1):
    cid = lax.axis_index("c")
    sid = lax.axis_index("s")
    wid = sid * NC + cid
    b = wid // TPB
    lbase = (wid % TPB) * RPT

    pltpu.sync_copy(idx_hbm.at[b], idxb_v)

    zeros = jnp.zeros((LANES,), jnp.float32)

    def zero_body(i, _):
        scale_v[pl.ds(i * LANES, LANES)] = zeros
        return 0

    lax.fori_loop(0, M // LANES, zero_body, 0)

    ones = jnp.ones((LANES,), jnp.float32)

    def hist_body(i, _):
        v = idxb_v[pl.ds(i * LANES, LANES)]
        plsc.addupdate_scatter(scale_v, [v], ones)
        return 0

    lax.fori_loop(0, L // LANES, hist_body, 0)

    def recip_body(i, _):
        sl = pl.ds(i * LANES, LANES)
        scale_v[sl] = 1.0 / scale_v[sl]
        return 0

    lax.fori_loop(0, M // LANES, recip_body, 0)

    def loc_body(c, _):
        for h in range(CHUNK // LANES):
            v = idxb_v[pl.ds(lbase + c * CHUNK + h * LANES, LANES)]
            gidx_v[c, pl.ds(h * LANES, LANES)] = v + b * M
            scl_v[pl.ds(c * CHUNK + h * LANES, LANES)] = plsc.load_gather(
                scale_v, [v])
        return 0

    lax.fori_loop(0, NCH, loc_body, 0)

    obase = wid * RPT

    def chunk_body(c, _):
        pltpu.async_copy(merged_hbm.at[gidx_v.at[c]], rows_v.at[0],
                         gsem0).wait()

        def row_body(j, _):
            idx16 = jnp.full((LANES,), c * CHUNK + j, jnp.int32)
            s = plsc.load_gather(scl_v, [idx16])
            for k in range(D // LANES):
                sl = pl.ds(k * LANES, LANES)
                rows_v[0, j, sl] = rows_v[0, j, sl] * s
            return 0

        lax.fori_loop(0, CHUNK, row_body, 0)
        pltpu.sync_copy(rows_v.at[0],
                        out_hbm.at[pl.ds(obase + c * CHUNK, CHUNK)])
        return 0

    lax.fori_loop(0, NCH, chunk_body, 0)


_unmerge_call = functools.partial(
    pl.kernel,
    out_type=jax.ShapeDtypeStruct((B * L, D), jnp.float32),
    mesh=plsc.VectorSubcoreMesh(core_axis_name="c", subcore_axis_name="s",
                                num_cores=NC, num_subcores=NS),
    scratch_types=[
        pltpu.VMEM((L,), jnp.int32),
        pltpu.VMEM((M,), jnp.float32),
        pltpu.VMEM((NCH, CHUNK), jnp.int32),
        pltpu.VMEM((RPT,), jnp.float32),
        pltpu.VMEM((2, CHUNK, D), jnp.float32),
        pltpu.SemaphoreType.DMA,
        pltpu.SemaphoreType.DMA,
        pltpu.SemaphoreType.DMA,
        pltpu.SemaphoreType.DMA,
    ],
    compiler_params=pltpu.CompilerParams(needs_layout_passes=False),
)(_unmerge_body)


@jax.jit
def kernel(merged_feats, source_maps):
    source_idx = source_maps[0]
    for i in range(1, source_maps.shape[0]):
        source_idx = jnp.take_along_axis(source_maps[i], source_idx, axis=1)
    assert merged_feats.shape == (B, M, D)
    assert source_idx.shape == (B, L)
    out = _unmerge_call(merged_feats.reshape(B * M, D),
                        source_idx.astype(jnp.int32))
    return out.reshape(B, L, D)


# speedup vs baseline: 19.5139x; 1.6404x over previous
import functools

import jax
import jax.numpy as jnp
from jax import lax
from jax.experimental import pallas as pl
from jax.experimental.pallas import tpu as pltpu
from jax.experimental.pallas import tpu_sc as plsc

B, M, L, D = 4, 4096, 8192, 1024
NC, NS, LANES = 2, 16, 16
NW = NC * NS
TPB = NW // B
RPT = L // TPB
CHUNK = 16
NCH = RPT // CHUNK
NBUF = 4


def _unmerge_body(merged_hbm, idx_hbm, out_hbm,
                  idxb_v, scale_v, gidx_v, scl_v, rows_v,
                  *sems):
    gsems, osems = sems[:NBUF], sems[NBUF:]
    cid = lax.axis_index("c")
    sid = lax.axis_index("s")
    wid = sid * NC + cid
    b = wid // TPB
    lbase = (wid % TPB) * RPT

    pltpu.sync_copy(idx_hbm.at[b], idxb_v)

    zeros = jnp.zeros((LANES,), jnp.float32)

    def zero_body(i, _):
        scale_v[pl.ds(i * LANES, LANES)] = zeros
        return 0

    lax.fori_loop(0, M // LANES, zero_body, 0)

    ones = jnp.ones((LANES,), jnp.float32)

    def hist_body(i, _):
        v = idxb_v[pl.ds(i * LANES, LANES)]
        plsc.addupdate_scatter(scale_v, [v], ones)
        return 0

    lax.fori_loop(0, L // LANES, hist_body, 0)

    def recip_body(i, _):
        sl = pl.ds(i * LANES, LANES)
        scale_v[sl] = 1.0 / scale_v[sl]
        return 0

    lax.fori_loop(0, M // LANES, recip_body, 0)

    def loc_body(c, _):
        for h in range(CHUNK // LANES):
            v = idxb_v[pl.ds(lbase + c * CHUNK + h * LANES, LANES)]
            gidx_v[c, pl.ds(h * LANES, LANES)] = v + b * M
            scl_v[pl.ds(c * CHUNK + h * LANES, LANES)] = plsc.load_gather(
                scale_v, [v])
        return 0

    lax.fori_loop(0, NCH, loc_body, 0)

    obase = wid * RPT

    def start_gather(c, buf):
        pltpu.async_copy(merged_hbm.at[gidx_v.at[c]], rows_v.at[buf],
                         gsems[buf])

    def wait_gather(c, buf):
        pltpu.make_async_copy(merged_hbm.at[gidx_v.at[c]], rows_v.at[buf],
                              gsems[buf]).wait()

    def start_out(c, buf):
        pltpu.async_copy(rows_v.at[buf],
                         out_hbm.at[pl.ds(obase + c * CHUNK, CHUNK)],
                         osems[buf])

    def wait_out(c, buf):
        pltpu.make_async_copy(rows_v.at[buf],
                              out_hbm.at[pl.ds(obase + c * CHUNK, CHUNK)],
                              osems[buf]).wait()

    def scale_chunk(c, buf):
        def row_body(j, _):
            idx16 = jnp.full((LANES,), c * CHUNK + j, jnp.int32)
            s = plsc.load_gather(scl_v, [idx16])
            for k in range(D // LANES):
                sl = pl.ds(k * LANES, LANES)
                rows_v[buf, j, sl] = rows_v[buf, j, sl] * s
            return 0

        lax.fori_loop(0, CHUNK, row_body, 0)

    start_gather(0, 0)
    start_gather(1, 1)

    def group_body(g, _):
        for buf in range(NBUF):
            c = g * NBUF + buf
            wait_gather(c, buf)
            scale_chunk(c, buf)
            start_out(c, buf)
            nb = (buf + 2) % NBUF
            nc = c + 2

            @pl.when(nc >= NBUF)
            def _():
                wait_out(nc - NBUF, nb)

            @pl.when(nc < NCH)
            def _():
                start_gather(nc, nb)
        return 0

    lax.fori_loop(0, NCH // NBUF, group_body, 0)

    wait_out(NCH - 2, (NCH - 2) % NBUF)
    wait_out(NCH - 1, (NCH - 1) % NBUF)


_unmerge_call = functools.partial(
    pl.kernel,
    out_type=jax.ShapeDtypeStruct((B * L, D), jnp.float32),
    mesh=plsc.VectorSubcoreMesh(core_axis_name="c", subcore_axis_name="s",
                                num_cores=NC, num_subcores=NS),
    scratch_types=[
        pltpu.VMEM((L,), jnp.int32),
        pltpu.VMEM((M,), jnp.float32),
        pltpu.VMEM((NCH, CHUNK), jnp.int32),
        pltpu.VMEM((RPT,), jnp.float32),
        pltpu.VMEM((NBUF, CHUNK, D), jnp.float32),
    ] + [pltpu.SemaphoreType.DMA] * (2 * NBUF),
    compiler_params=pltpu.CompilerParams(needs_layout_passes=False),
)(_unmerge_body)


@jax.jit
def kernel(merged_feats, source_maps):
    source_idx = source_maps[0]
    for i in range(1, source_maps.shape[0]):
        source_idx = jnp.take_along_axis(source_maps[i], source_idx, axis=1)
    assert merged_feats.shape == (B, M, D)
    assert source_idx.shape == (B, L)
    out = _unmerge_call(merged_feats.reshape(B * M, D),
                        source_idx.astype(jnp.int32))
    return out.reshape(B, L, D)


# speedup vs baseline: 19.6713x; 1.0081x over previous
import functools

import jax
import jax.numpy as jnp
from jax import lax
from jax.experimental import pallas as pl
from jax.experimental.pallas import tpu as pltpu
from jax.experimental.pallas import tpu_sc as plsc

B, M, L, D = 4, 4096, 8192, 1024
NC, NS, LANES = 2, 16, 16
NW = NC * NS
TPB = NW // B
RPT = L // TPB
CHUNK = 16
NCH = RPT // CHUNK
NBUF = 4
PF = 2


def _unmerge_body(merged_hbm, idx_hbm, out_hbm,
                  idxb_v, scale_v, gidx_v, scl_v, rows_v,
                  *sems):
    gsems, osems = sems[:NBUF], sems[NBUF:]
    cid = lax.axis_index("c")
    sid = lax.axis_index("s")
    wid = sid * NC + cid
    b = wid // TPB
    lbase = (wid % TPB) * RPT

    pltpu.sync_copy(idx_hbm.at[b], idxb_v)

    obase = wid * RPT

    def gidx_body(c, _):
        for h in range(CHUNK // LANES):
            v = idxb_v[pl.ds(lbase + c * CHUNK + h * LANES, LANES)]
            gidx_v[c, pl.ds(h * LANES, LANES)] = v + b * M
        return 0

    lax.fori_loop(0, NCH, gidx_body, 0)

    def start_gather(c, buf):
        pltpu.async_copy(merged_hbm.at[gidx_v.at[c]], rows_v.at[buf],
                         gsems[buf])

    for c0 in range(PF):
        start_gather(c0, c0)

    zeros = jnp.zeros((LANES,), jnp.float32)

    def zero_body(i, _):
        scale_v[pl.ds(i * LANES, LANES)] = zeros
        return 0

    lax.fori_loop(0, M // LANES, zero_body, 0)

    ones = jnp.ones((LANES,), jnp.float32)

    def hist_body(i, _):
        v = idxb_v[pl.ds(i * LANES, LANES)]
        plsc.addupdate_scatter(scale_v, [v], ones)
        return 0

    lax.fori_loop(0, L // LANES, hist_body, 0)

    def recip_body(i, _):
        sl = pl.ds(i * LANES, LANES)
        scale_v[sl] = 1.0 / scale_v[sl]
        return 0

    lax.fori_loop(0, M // LANES, recip_body, 0)

    def scl_body(j, _):
        v = idxb_v[pl.ds(lbase + j * LANES, LANES)]
        scl_v[pl.ds(j * LANES, LANES)] = plsc.load_gather(scale_v, [v])
        return 0

    lax.fori_loop(0, RPT // LANES, scl_body, 0)

    def wait_gather(c, buf):
        pltpu.make_async_copy(merged_hbm.at[gidx_v.at[c]], rows_v.at[buf],
                              gsems[buf]).wait()

    def start_out(c, buf):
        pltpu.async_copy(rows_v.at[buf],
                         out_hbm.at[pl.ds(obase + c * CHUNK, CHUNK)],
                         osems[buf])

    def wait_out(c, buf):
        pltpu.make_async_copy(rows_v.at[buf],
                              out_hbm.at[pl.ds(obase + c * CHUNK, CHUNK)],
                              osems[buf]).wait()

    def scale_chunk(c, buf):
        def row_body(j, _):
            idx16 = jnp.full((LANES,), c * CHUNK + j, jnp.int32)
            s = plsc.load_gather(scl_v, [idx16])
            for k in range(D // LANES):
                sl = pl.ds(k * LANES, LANES)
                rows_v[buf, j, sl] = rows_v[buf, j, sl] * s
            return 0

        lax.fori_loop(0, CHUNK, row_body, 0)

    def group_body(g, _):
        for buf in range(NBUF):
            c = g * NBUF + buf
            wait_gather(c, buf)
            scale_chunk(c, buf)
            start_out(c, buf)
            nc = c + PF
            nb = (buf + PF) % NBUF

            @pl.when(nc >= NBUF)
            def _():
                wait_out(nc - NBUF, nb)

            @pl.when(nc < NCH)
            def _():
                start_gather(nc, nb)
        return 0

    lax.fori_loop(0, NCH // NBUF, group_body, 0)

    for c0 in range(NCH - (NBUF - PF), NCH):
        wait_out(c0, c0 % NBUF)


_unmerge_call = functools.partial(
    pl.kernel,
    out_type=jax.ShapeDtypeStruct((B * L, D), jnp.float32),
    mesh=plsc.VectorSubcoreMesh(core_axis_name="c", subcore_axis_name="s",
                                num_cores=NC, num_subcores=NS),
    scratch_types=[
        pltpu.VMEM((L,), jnp.int32),
        pltpu.VMEM((M,), jnp.float32),
        pltpu.VMEM((NCH, CHUNK), jnp.int32),
        pltpu.VMEM((RPT,), jnp.float32),
        pltpu.VMEM((NBUF, CHUNK, D), jnp.float32),
    ] + [pltpu.SemaphoreType.DMA] * (2 * NBUF),
    compiler_params=pltpu.CompilerParams(needs_layout_passes=False),
)(_unmerge_body)


@jax.jit
def kernel(merged_feats, source_maps):
    source_idx = source_maps[0]
    for i in range(1, source_maps.shape[0]):
        source_idx = jnp.take_along_axis(source_maps[i], source_idx, axis=1)
    assert merged_feats.shape == (B, M, D)
    assert source_idx.shape == (B, L)
    out = _unmerge_call(merged_feats.reshape(B * M, D),
                        source_idx.astype(jnp.int32))
    return out.reshape(B, L, D)


# speedup vs baseline: 21.1074x; 1.0730x over previous
import functools

import jax
import jax.numpy as jnp
from jax import lax
from jax.experimental import pallas as pl
from jax.experimental.pallas import tpu as pltpu
from jax.experimental.pallas import tpu_sc as plsc

B, M, L, D = 4, 4096, 8192, 1024
NC, NS, LANES = 2, 16, 16
NW = NC * NS
TPB = NW // B
RPT = L // TPB
CHUNK = 16
NCH = RPT // CHUNK
NBUF = 4
PF = 3


def _unmerge_body(merged_hbm, idx_hbm, out_hbm,
                  idxb_v, scale_v, gidx_v, scl_v, rows_v,
                  *sems):
    gsems, osems = sems[:NBUF], sems[NBUF:]
    cid = lax.axis_index("c")
    sid = lax.axis_index("s")
    wid = sid * NC + cid
    b = wid // TPB
    lbase = (wid % TPB) * RPT

    pltpu.sync_copy(idx_hbm.at[b], idxb_v)

    obase = wid * RPT

    def gidx_body(c, _):
        for h in range(CHUNK // LANES):
            v = idxb_v[pl.ds(lbase + c * CHUNK + h * LANES, LANES)]
            gidx_v[c, pl.ds(h * LANES, LANES)] = v + b * M
        return 0

    lax.fori_loop(0, NCH, gidx_body, 0)

    def start_gather(c, buf):
        pltpu.async_copy(merged_hbm.at[gidx_v.at[c]], rows_v.at[buf],
                         gsems[buf])

    for c0 in range(PF):
        start_gather(c0, c0)

    zeros = jnp.zeros((LANES,), jnp.float32)

    def zero_body(i, _):
        scale_v[pl.ds(i * LANES, LANES)] = zeros
        return 0

    lax.fori_loop(0, M // LANES, zero_body, 0)

    ones = jnp.ones((LANES,), jnp.float32)

    def hist_body(i, _):
        v = idxb_v[pl.ds(i * LANES, LANES)]
        plsc.addupdate_scatter(scale_v, [v], ones)
        return 0

    lax.fori_loop(0, L // LANES, hist_body, 0)

    def recip_body(i, _):
        sl = pl.ds(i * LANES, LANES)
        scale_v[sl] = 1.0 / scale_v[sl]
        return 0

    lax.fori_loop(0, M // LANES, recip_body, 0)

    def scl_body(j, _):
        v = idxb_v[pl.ds(lbase + j * LANES, LANES)]
        scl_v[pl.ds(j * LANES, LANES)] = plsc.load_gather(scale_v, [v])
        return 0

    lax.fori_loop(0, RPT // LANES, scl_body, 0)

    def wait_gather(c, buf):
        pltpu.make_async_copy(merged_hbm.at[gidx_v.at[c]], rows_v.at[buf],
                              gsems[buf]).wait()

    def start_out(c, buf):
        pltpu.async_copy(rows_v.at[buf],
                         out_hbm.at[pl.ds(obase + c * CHUNK, CHUNK)],
                         osems[buf])

    def wait_out(c, buf):
        pltpu.make_async_copy(rows_v.at[buf],
                              out_hbm.at[pl.ds(obase + c * CHUNK, CHUNK)],
                              osems[buf]).wait()

    def scale_chunk(c, buf):
        def row_body(j, _):
            idx16 = jnp.full((LANES,), c * CHUNK + j, jnp.int32)
            s = plsc.load_gather(scl_v, [idx16])
            for k in range(D // LANES):
                sl = pl.ds(k * LANES, LANES)
                rows_v[buf, j, sl] = rows_v[buf, j, sl] * s
            return 0

        lax.fori_loop(0, CHUNK, row_body, 0)

    def group_body(g, _):
        for buf in range(NBUF):
            c = g * NBUF + buf
            wait_gather(c, buf)
            scale_chunk(c, buf)
            start_out(c, buf)
            nc = c + PF
            nb = (buf + PF) % NBUF

            @pl.when(nc >= NBUF)
            def _():
                wait_out(nc - NBUF, nb)

            @pl.when(nc < NCH)
            def _():
                start_gather(nc, nb)
        return 0

    lax.fori_loop(0, NCH // NBUF, group_body, 0)

    for c0 in range(NCH - (NBUF - PF), NCH):
        wait_out(c0, c0 % NBUF)


_unmerge_call = functools.partial(
    pl.kernel,
    out_type=jax.ShapeDtypeStruct((B * L, D), jnp.float32),
    mesh=plsc.VectorSubcoreMesh(core_axis_name="c", subcore_axis_name="s",
                                num_cores=NC, num_subcores=NS),
    scratch_types=[
        pltpu.VMEM((L,), jnp.int32),
        pltpu.VMEM((M,), jnp.float32),
        pltpu.VMEM((NCH, CHUNK), jnp.int32),
        pltpu.VMEM((RPT,), jnp.float32),
        pltpu.VMEM((NBUF, CHUNK, D), jnp.float32),
    ] + [pltpu.SemaphoreType.DMA] * (2 * NBUF),
    compiler_params=pltpu.CompilerParams(needs_layout_passes=False),
)(_unmerge_body)


@jax.jit
def kernel(merged_feats, source_maps):
    source_idx = source_maps[0]
    for i in range(1, source_maps.shape[0]):
        source_idx = jnp.take_along_axis(source_maps[i], source_idx, axis=1)
    assert merged_feats.shape == (B, M, D)
    assert source_idx.shape == (B, L)
    out = _unmerge_call(merged_feats.reshape(B * M, D),
                        source_idx.astype(jnp.int32))
    return out.reshape(B, L, D)
